# Initial kernel scaffold; baseline (speedup 1.0000x reference)
import jax
import jax.numpy as jnp
from jax.experimental import pallas as pl


def kernel(x, position_table):
    raise NotImplementedError("write your pallas kernel here")


# speedup vs baseline: 1.2345x; 1.2345x over previous
import jax
import jax.numpy as jnp
from jax.experimental import pallas as pl


def _concat_body(x_ref, pos_ref, o_ref):
    xb = x_ref[...]
    pos = jnp.broadcast_to(pos_ref[...][None], xb.shape)
    o_ref[...] = jnp.concatenate([xb, pos], axis=-1)


def kernel(x, position_table):
    B, L, D = x.shape
    pos = position_table[:L]
    BB = 128
    return pl.pallas_call(
        _concat_body,
        grid=(B // BB,),
        in_specs=[
            pl.BlockSpec((BB, L, D), lambda i: (i, 0, 0)),
            pl.BlockSpec((L, D), lambda i: (0, 0)),
        ],
        out_specs=pl.BlockSpec((BB, L, 2 * D), lambda i: (i, 0, 0)),
        out_shape=jax.ShapeDtypeStruct((B, L, 2 * D), x.dtype),
    )(x, pos)
